# baseline (speedup 1.0000x reference)
import functools

import jax
import jax.numpy as jnp
from jax import lax
from jax.experimental import pallas as pl
from jax.experimental.pallas import tpu as pltpu
from jax.experimental.pallas import tpu_sc as plsc

N_NODES = 10000
N_EDGES = 320000
NFEAT = 128
HIDDEN = 256
NCLASS = 6

NC = 2
NS = 16
NW = NC * NS
CHUNK = 128
EDGES_PER_W = -(-N_EDGES // NW)
NCHUNK = 2 * (-(-EDGES_PER_W // (2 * CHUNK)))
EP = NW * NCHUNK * CHUNK
TRASH = N_NODES
ACC_ROWS = 10240
ZPT = ACC_ROWS // NS
OPT = 640
OLAST = N_NODES - OPT * (NS - 1)

_f32 = jnp.float32


def _copy_out_share(s, c, acc, out):
    @pl.when(s < NS - 1)
    def _():
        pltpu.sync_copy(acc.at[pl.ds(s * OPT, OPT)],
                        out.at[c, pl.ds(s * OPT, OPT)])

    @pl.when(s == NS - 1)
    def _():
        off = OPT * (NS - 1)
        pltpu.sync_copy(acc.at[pl.ds(off, OLAST)],
                        out.at[c, pl.ds(off, OLAST)])


def _sc_mesh():
    return plsc.VectorSubcoreMesh(
        core_axis_name="c", subcore_axis_name="s",
        num_cores=NC, num_subcores=NS)


def _make_deg_kernel():
    @functools.partial(
        pl.kernel,
        out_type=jax.ShapeDtypeStruct((NC, N_NODES, 16), _f32),
        mesh=_sc_mesh(),
        scratch_types=[
            pltpu.VMEM((NCHUNK, CHUNK), jnp.int32),
            pltpu.VMEM((CHUNK, 16), _f32),
            pltpu.VMEM_SHARED((ACC_ROWS, 16), _f32),
        ],
        compiler_params=pltpu.CompilerParams(use_tc_tiling_on_sc=False),
    )
    def deg_kernel(dstp, out, dst_v, buf_v, acc):
        c = lax.axis_index("c")
        s = lax.axis_index("s")
        wid = s * NC + c

        @pl.loop(0, CHUNK)
        def _zero_buf(i):
            buf_v[i, :] = jnp.zeros((16,), _f32)

        for k in range(ZPT // CHUNK):
            pltpu.sync_copy(buf_v, acc.at[pl.ds(s * ZPT + k * CHUNK, CHUNK)])

        pltpu.sync_copy(dstp.at[wid], dst_v)

        @pl.loop(0, CHUNK)
        def _fill_ones(i):
            buf_v[i, :] = jnp.ones((16,), _f32)

        plsc.subcore_barrier()

        @pl.loop(0, NCHUNK)
        def _scatter(j):
            pltpu.sync_copy(buf_v, acc.at[dst_v.at[j]], add=True)

        plsc.subcore_barrier()
        _copy_out_share(s, c, acc, out)

    return deg_kernel


def _make_agg_kernel(width):
    @functools.partial(
        pl.kernel,
        out_type=jax.ShapeDtypeStruct((NC, N_NODES, width), _f32),
        mesh=_sc_mesh(),
        scratch_types=[
            pltpu.VMEM((NCHUNK // 2, CHUNK), jnp.int32),
            pltpu.VMEM((NCHUNK // 2, CHUNK), jnp.int32),
            pltpu.VMEM((CHUNK, width), _f32),
            pltpu.VMEM((CHUNK, width), _f32),
            pltpu.VMEM_SHARED((ACC_ROWS, width), _f32),
            pltpu.SemaphoreType.DMA,
            pltpu.SemaphoreType.DMA,
            pltpu.SemaphoreType.DMA,
        ],
        compiler_params=pltpu.CompilerParams(use_tc_tiling_on_sc=False),
    )
    def agg_kernel(table, srcp, dstp, out, src_v, dst_v, rows0, rows1, acc,
                   sem_g, sem_s0, sem_s1):
        c = lax.axis_index("c")
        s = lax.axis_index("s")
        wid = s * NC + c
        half = NCHUNK // 2

        @pl.loop(0, CHUNK)
        def _zero_buf(i):
            for j in range(width // 16):
                rows0[i, pl.ds(j * 16, 16)] = jnp.zeros((16,), _f32)

        for k in range(ZPT // CHUNK):
            pltpu.sync_copy(rows0, acc.at[pl.ds(s * ZPT + k * CHUNK, CHUNK)])

        plsc.subcore_barrier()

        def start_g(j, buf):
            pltpu.async_copy(table.at[src_v.at[j]], buf, sem_g)

        def wait_g(j, buf):
            pltpu.make_async_copy(table.at[src_v.at[j]], buf, sem_g).wait()

        def start_s(j, buf, sem):
            pltpu.async_copy(buf, acc.at[dst_v.at[j]], sem, add=True)

        def wait_s(j, buf, sem):
            pltpu.make_async_copy(buf, acc.at[dst_v.at[j]], sem).wait()

        def do_half(base):
            pltpu.sync_copy(srcp.at[wid, pl.ds(base, half)], src_v)
            pltpu.sync_copy(dstp.at[wid, pl.ds(base, half)], dst_v)

            @pl.loop(0, half)
            def _edge_chunk(j):
                pltpu.async_copy(table.at[src_v.at[j]], rows0, sem_g).wait()
                pltpu.sync_copy(rows0, acc.at[dst_v.at[j]], add=True)

        do_half(0)
        do_half(half)
        plsc.subcore_barrier()
        _copy_out_share(s, c, acc, out)

    return agg_kernel


_TCB = 1000


def _tc_prep(d0, d1, x):
    def body(d0_ref, d1_ref, x_ref, xs_ref, db_ref):
        deg = d0_ref[...] + d1_ref[...] + 1.0
        dinv = lax.rsqrt(deg)
        db = jnp.broadcast_to(dinv[:, 0:1], (_TCB, NFEAT))
        db_ref[...] = db
        xs_ref[...] = x_ref[...] * db

    return pl.pallas_call(
        body,
        grid=(N_NODES // _TCB,),
        in_specs=[
            pl.BlockSpec((_TCB, 16), lambda i: (i, 0)),
            pl.BlockSpec((_TCB, 16), lambda i: (i, 0)),
            pl.BlockSpec((_TCB, NFEAT), lambda i: (i, 0)),
        ],
        out_specs=[
            pl.BlockSpec((_TCB, NFEAT), lambda i: (i, 0)),
            pl.BlockSpec((_TCB, NFEAT), lambda i: (i, 0)),
        ],
        out_shape=[
            jax.ShapeDtypeStruct((N_NODES, NFEAT), _f32),
            jax.ShapeDtypeStruct((N_NODES, NFEAT), _f32),
        ],
    )(d0, d1, x)


def _tc_mid(s0, s1, xs, db, W1, b1r, W2p):
    def body(s0_ref, s1_ref, xs_ref, db_ref, w1_ref, b1_ref, w2_ref, ps_ref):
        db = db_ref[...]
        z = (s0_ref[...] + s1_ref[...] + xs_ref[...]) * db
        h = jnp.dot(z, w1_ref[...], preferred_element_type=_f32,
                    precision=lax.Precision.HIGHEST) + b1_ref[...]
        h = jnp.maximum(h, 0.0)
        p = jnp.dot(h, w2_ref[...], preferred_element_type=_f32,
                    precision=lax.Precision.HIGHEST)
        ps_ref[...] = p * db[:, :16]

    return pl.pallas_call(
        body,
        grid=(N_NODES // _TCB,),
        in_specs=[
            pl.BlockSpec((_TCB, NFEAT), lambda i: (i, 0)),
            pl.BlockSpec((_TCB, NFEAT), lambda i: (i, 0)),
            pl.BlockSpec((_TCB, NFEAT), lambda i: (i, 0)),
            pl.BlockSpec((_TCB, NFEAT), lambda i: (i, 0)),
            pl.BlockSpec((NFEAT, HIDDEN), lambda i: (0, 0)),
            pl.BlockSpec((1, HIDDEN), lambda i: (0, 0)),
            pl.BlockSpec((HIDDEN, 16), lambda i: (0, 0)),
        ],
        out_specs=pl.BlockSpec((_TCB, 16), lambda i: (i, 0)),
        out_shape=jax.ShapeDtypeStruct((N_NODES, 16), _f32),
    )(s0, s1, xs, db, W1, b1r, W2p)


def _tc_final(t0, t1, ps, db16, b2r):
    def body(t0_ref, t1_ref, ps_ref, db_ref, b2_ref, o_ref):
        o_ref[...] = ((t0_ref[...] + t1_ref[...] + ps_ref[...])
                      * db_ref[...] + b2_ref[...])

    return pl.pallas_call(
        body,
        grid=(N_NODES // _TCB,),
        in_specs=[
            pl.BlockSpec((_TCB, 16), lambda i: (i, 0)),
            pl.BlockSpec((_TCB, 16), lambda i: (i, 0)),
            pl.BlockSpec((_TCB, 16), lambda i: (i, 0)),
            pl.BlockSpec((_TCB, 16), lambda i: (i, 0)),
            pl.BlockSpec((1, 16), lambda i: (0, 0)),
        ],
        out_specs=pl.BlockSpec((_TCB, 16), lambda i: (i, 0)),
        out_shape=jax.ShapeDtypeStruct((N_NODES, 16), _f32),
    )(t0, t1, ps, db16, b2r)


def kernel(x, edge_index, W1, b1, W2, b2):
    src = edge_index[0].astype(jnp.int32)
    dst = edge_index[1].astype(jnp.int32)
    pad = EP - N_EDGES
    srcp = jnp.concatenate([src, jnp.zeros((pad,), jnp.int32)])
    srcp = srcp.reshape(NW, NCHUNK, CHUNK)
    dstp = jnp.concatenate([dst, jnp.full((pad,), TRASH, jnp.int32)])
    dstp = dstp.reshape(NW, NCHUNK, CHUNK)

    degp = _make_deg_kernel()(dstp)
    xs, db = _tc_prep(degp[0], degp[1], x)
    S = _make_agg_kernel(NFEAT)(xs, srcp, dstp)

    W2p = jnp.zeros((HIDDEN, 16), _f32).at[:, :NCLASS].set(W2)
    ps = _tc_mid(S[0], S[1], xs, db, W1,
                 b1.reshape(1, HIDDEN), W2p)
    T = _make_agg_kernel(16)(ps, srcp, dstp)

    b2r = jnp.zeros((1, 16), _f32).at[0, :NCLASS].set(b2)
    out16 = _tc_final(T[0], T[1], ps, db[:, :16], b2r)
    return out16[:, :NCLASS]


# speedup vs baseline: 1.3483x; 1.3483x over previous
import functools

import jax
import jax.numpy as jnp
from jax import lax
from jax.experimental import pallas as pl
from jax.experimental.pallas import tpu as pltpu
from jax.experimental.pallas import tpu_sc as plsc

N_NODES = 10000
N_EDGES = 320000
NFEAT = 128
HIDDEN = 256
NCLASS = 6

NC = 2
NS = 16
NW = NC * NS
CHUNK = 128
EDGES_PER_W = -(-N_EDGES // NW)
NCHUNK = -(-EDGES_PER_W // CHUNK)
EP = NW * NCHUNK * CHUNK
TRASH = N_NODES
ACC_ROWS = 10240
ZPT = ACC_ROWS // NS
OPT = 640
OLAST = N_NODES - OPT * (NS - 1)

_f32 = jnp.float32


def _copy_out_share(s, c, acc, out):
    @pl.when(s < NS - 1)
    def _():
        pltpu.sync_copy(acc.at[pl.ds(s * OPT, OPT)],
                        out.at[c, pl.ds(s * OPT, OPT)])

    @pl.when(s == NS - 1)
    def _():
        off = OPT * (NS - 1)
        pltpu.sync_copy(acc.at[pl.ds(off, OLAST)],
                        out.at[c, pl.ds(off, OLAST)])


def _sc_mesh():
    return plsc.VectorSubcoreMesh(
        core_axis_name="c", subcore_axis_name="s",
        num_cores=NC, num_subcores=NS)


def _make_deg_kernel():
    @functools.partial(
        pl.kernel,
        out_type=jax.ShapeDtypeStruct((NC, N_NODES, 16), _f32),
        mesh=_sc_mesh(),
        scratch_types=[
            pltpu.VMEM((NCHUNK, CHUNK), jnp.int32),
            pltpu.VMEM((CHUNK, 16), _f32),
            pltpu.VMEM_SHARED((ACC_ROWS, 16), _f32),
        ],
        compiler_params=pltpu.CompilerParams(use_tc_tiling_on_sc=False),
    )
    def deg_kernel(dstp, out, dst_v, buf_v, acc):
        c = lax.axis_index("c")
        s = lax.axis_index("s")
        wid = s * NC + c

        @pl.loop(0, CHUNK)
        def _zero_buf(i):
            buf_v[i, :] = jnp.zeros((16,), _f32)

        for k in range(ZPT // CHUNK):
            pltpu.sync_copy(buf_v, acc.at[pl.ds(s * ZPT + k * CHUNK, CHUNK)])

        pltpu.sync_copy(dstp.at[wid], dst_v)

        @pl.loop(0, CHUNK)
        def _fill_ones(i):
            buf_v[i, :] = jnp.ones((16,), _f32)

        plsc.subcore_barrier()

        @pl.loop(0, NCHUNK)
        def _scatter(j):
            pltpu.sync_copy(buf_v, acc.at[dst_v.at[j]], add=True)

        plsc.subcore_barrier()
        _copy_out_share(s, c, acc, out)

    return deg_kernel


def _make_agg_kernel(width):
    @functools.partial(
        pl.kernel,
        out_type=jax.ShapeDtypeStruct((NC, N_NODES, width), _f32),
        mesh=_sc_mesh(),
        scratch_types=[
            pltpu.VMEM((NCHUNK, CHUNK), jnp.int32),
            pltpu.VMEM((NCHUNK, CHUNK), jnp.int32),
            pltpu.VMEM((CHUNK, width), _f32),
            pltpu.VMEM_SHARED((ACC_ROWS, width), _f32),
            pltpu.SemaphoreType.DMA,
        ],
        compiler_params=pltpu.CompilerParams(use_tc_tiling_on_sc=False),
    )
    def agg_kernel(table, srcp, dstp, out, src_v, dst_v, rows_v, acc, sem):
        c = lax.axis_index("c")
        s = lax.axis_index("s")
        wid = s * NC + c

        @pl.loop(0, CHUNK)
        def _zero_buf(i):
            for j in range(width // 16):
                rows_v[i, pl.ds(j * 16, 16)] = jnp.zeros((16,), _f32)

        for k in range(ZPT // CHUNK):
            pltpu.sync_copy(rows_v, acc.at[pl.ds(s * ZPT + k * CHUNK, CHUNK)])

        pltpu.sync_copy(srcp.at[wid], src_v)
        pltpu.sync_copy(dstp.at[wid], dst_v)
        plsc.subcore_barrier()

        @pl.loop(0, NCHUNK)
        def _edge_chunk(j):
            pltpu.async_copy(table.at[src_v.at[j]], rows_v, sem).wait()
            pltpu.sync_copy(rows_v, acc.at[dst_v.at[j]], add=True)

        plsc.subcore_barrier()
        _copy_out_share(s, c, acc, out)

    return agg_kernel


_TCB = 1000


def _tc_prep(d0, d1, x):
    def body(d0_ref, d1_ref, x_ref, xs_ref, db_ref):
        deg = d0_ref[...] + d1_ref[...] + 1.0
        dinv = lax.rsqrt(deg)
        db = jnp.broadcast_to(dinv[:, 0:1], (_TCB, NFEAT))
        db_ref[...] = db
        xs_ref[...] = x_ref[...] * db

    return pl.pallas_call(
        body,
        grid=(N_NODES // _TCB,),
        in_specs=[
            pl.BlockSpec((_TCB, 16), lambda i: (i, 0)),
            pl.BlockSpec((_TCB, 16), lambda i: (i, 0)),
            pl.BlockSpec((_TCB, NFEAT), lambda i: (i, 0)),
        ],
        out_specs=[
            pl.BlockSpec((_TCB, NFEAT), lambda i: (i, 0)),
            pl.BlockSpec((_TCB, NFEAT), lambda i: (i, 0)),
        ],
        out_shape=[
            jax.ShapeDtypeStruct((N_NODES, NFEAT), _f32),
            jax.ShapeDtypeStruct((N_NODES, NFEAT), _f32),
        ],
    )(d0, d1, x)


def _tc_mid(s0, s1, xs, db, W1, b1r, W2p):
    def body(s0_ref, s1_ref, xs_ref, db_ref, w1_ref, b1_ref, w2_ref, ps_ref):
        db = db_ref[...]
        z = (s0_ref[...] + s1_ref[...] + xs_ref[...]) * db
        h = jnp.dot(z, w1_ref[...], preferred_element_type=_f32,
                    precision=lax.Precision.HIGHEST) + b1_ref[...]
        h = jnp.maximum(h, 0.0)
        p = jnp.dot(h, w2_ref[...], preferred_element_type=_f32,
                    precision=lax.Precision.HIGHEST)
        ps_ref[...] = p * db[:, :16]

    return pl.pallas_call(
        body,
        grid=(N_NODES // _TCB,),
        in_specs=[
            pl.BlockSpec((_TCB, NFEAT), lambda i: (i, 0)),
            pl.BlockSpec((_TCB, NFEAT), lambda i: (i, 0)),
            pl.BlockSpec((_TCB, NFEAT), lambda i: (i, 0)),
            pl.BlockSpec((_TCB, NFEAT), lambda i: (i, 0)),
            pl.BlockSpec((NFEAT, HIDDEN), lambda i: (0, 0)),
            pl.BlockSpec((1, HIDDEN), lambda i: (0, 0)),
            pl.BlockSpec((HIDDEN, 16), lambda i: (0, 0)),
        ],
        out_specs=pl.BlockSpec((_TCB, 16), lambda i: (i, 0)),
        out_shape=jax.ShapeDtypeStruct((N_NODES, 16), _f32),
    )(s0, s1, xs, db, W1, b1r, W2p)


def _tc_final(t0, t1, ps, db16, b2r):
    def body(t0_ref, t1_ref, ps_ref, db_ref, b2_ref, o_ref):
        o_ref[...] = ((t0_ref[...] + t1_ref[...] + ps_ref[...])
                      * db_ref[...] + b2_ref[...])

    return pl.pallas_call(
        body,
        grid=(N_NODES // _TCB,),
        in_specs=[
            pl.BlockSpec((_TCB, 16), lambda i: (i, 0)),
            pl.BlockSpec((_TCB, 16), lambda i: (i, 0)),
            pl.BlockSpec((_TCB, 16), lambda i: (i, 0)),
            pl.BlockSpec((_TCB, 16), lambda i: (i, 0)),
            pl.BlockSpec((1, 16), lambda i: (0, 0)),
        ],
        out_specs=pl.BlockSpec((_TCB, 16), lambda i: (i, 0)),
        out_shape=jax.ShapeDtypeStruct((N_NODES, 16), _f32),
    )(t0, t1, ps, db16, b2r)


def kernel(x, edge_index, W1, b1, W2, b2):
    src = edge_index[0].astype(jnp.int32)
    dst = edge_index[1].astype(jnp.int32)
    pad = EP - N_EDGES
    srcp = jnp.concatenate([src, jnp.zeros((pad,), jnp.int32)])
    srcp = srcp.reshape(NW, NCHUNK, CHUNK)
    dstp = jnp.concatenate([dst, jnp.full((pad,), TRASH, jnp.int32)])
    dstp = dstp.reshape(NW, NCHUNK, CHUNK)

    degp = _make_deg_kernel()(dstp)
    xs, db = _tc_prep(degp[0], degp[1], x)
    S = _make_agg_kernel(NFEAT)(xs, srcp, dstp)

    W2p = jnp.zeros((HIDDEN, 16), _f32).at[:, :NCLASS].set(W2)
    ps = _tc_mid(S[0], S[1], xs, db, W1,
                 b1.reshape(1, HIDDEN), W2p)
    T = _make_agg_kernel(16)(ps, srcp, dstp)

    b2r = jnp.zeros((1, 16), _f32).at[0, :NCLASS].set(b2)
    out16 = _tc_final(T[0], T[1], ps, db[:, :16], b2r)
    return out16[:, :NCLASS]
